# baseline (speedup 1.0000x reference)
from functools import partial

import numpy as np
import jax
import jax.numpy as jnp
from jax import lax
from jax.experimental import pallas as pl
from jax.experimental.pallas import tpu as pltpu

B, L, H, D = 2, 2048, 12, 64
U = 40
SCALE = 0.125
NEG = -1e30
HPS = 2


def _threefry2x32(k1, k2, x0, x1):
    def rotl(x, d):
        return ((x << np.uint32(d)) | (x >> np.uint32(32 - d))).astype(np.uint32)
    x0 = x0.astype(np.uint32).copy()
    x1 = x1.astype(np.uint32).copy()
    ks = [np.uint32(k1), np.uint32(k2),
          np.uint32(np.uint32(k1) ^ np.uint32(k2) ^ np.uint32(0x1BD11BDA))]
    R = [(13, 15, 26, 6), (17, 29, 16, 24)]
    x0 = (x0 + ks[0]).astype(np.uint32)
    x1 = (x1 + ks[1]).astype(np.uint32)
    for i in range(5):
        for r in R[i % 2]:
            x0 = (x0 + x1).astype(np.uint32)
            x1 = rotl(x1, r) ^ x0
        x0 = (x0 + ks[(i + 1) % 3]).astype(np.uint32)
        x1 = (x1 + ks[(i + 2) % 3] + np.uint32(i + 1)).astype(np.uint32)
    return x0, x1


def _build_cnt_t() -> np.ndarray:
    b1, b2 = _threefry2x32(0, 42, np.zeros(2, np.uint32),
                           np.arange(2, dtype=np.uint32))
    lo1, lo2 = _threefry2x32(b1[1], b2[1], np.zeros(L * U, np.uint32),
                             np.arange(L * U, dtype=np.uint32))
    idx = ((lo1 ^ lo2) % np.uint32(L)).astype(np.int64).reshape(L, U)
    cnt_t = np.zeros((L, L), np.int8)
    np.add.at(cnt_t, (idx, np.broadcast_to(np.arange(L)[:, None], (L, U))), 1)
    return cnt_t


_CNT_T = _build_cnt_t()


def _body(q_ref, k_ref, v_ref, pek_ref, cnt_ref, uw_ref, vw_ref, ub_ref,
          vb_ref, out_ref, oh_ref):
    f32 = jnp.float32
    dot = partial(lax.dot_general, preferred_element_type=f32)
    cnt = cnt_ref[...].astype(f32)

    Ms = []
    for hh in range(HPS):
        sl = slice(hh * D, (hh + 1) * D)
        K1 = k_ref[:, sl]
        Q1 = q_ref[:, sl]
        Kh = K1.astype(jnp.bfloat16)
        Qh = Q1.astype(jnp.bfloat16)
        Kl = (K1 - Kh.astype(f32)).astype(jnp.bfloat16)
        Ql = (Q1 - Qh.astype(f32)).astype(jnp.bfloat16)
        cd = (((1,), (1,)), ((), ()))
        St = dot(Kh, Qh, cd) + dot(Kh, Ql, cd) + dot(Kl, Qh, cd)
        smax = jnp.max(jnp.where(cnt > 0.0, St, NEG), axis=0, keepdims=True)
        ssum = jnp.sum(St * cnt, axis=0, keepdims=True)
        Ms.append(smax - ssum * (1.0 / L))
    M0 = jnp.concatenate(Ms, axis=0)

    iota = lax.broadcasted_iota(jnp.int32, (HPS, L), 1)

    def topk_body(i, Mv):
        maxv = jnp.max(Mv, axis=1, keepdims=True)
        idx = jnp.min(jnp.where(Mv == maxv, iota, L), axis=1, keepdims=True)
        hit = iota == idx
        oh_ref[:, pl.ds(i, 1), :] = hit.astype(f32)[:, None, :]
        return jnp.where(hit, NEG, Mv)

    lax.fori_loop(0, U, topk_body, M0)

    bias = ub_ref[...] + vb_ref[...]
    for hh in range(HPS):
        sl = slice(hh * D, (hh + 1) * D)
        Q = q_ref[:, sl]
        K = k_ref[:, sl]
        V = v_ref[:, sl]
        PEK = pek_ref[:, sl]
        OH = oh_ref[hh]
        Qr = dot(OH, Q, (((1,), (0,)), ((), ())))
        G = dot(K + PEK, Qr, (((1,), (1,)), ((), ())))
        G = G + dot(K, uw_ref[...], (((1,), (1,)), ((), ())))
        G = G + dot(PEK, vw_ref[...], (((1,), (1,)), ((), ())))
        G = (G + bias) * SCALE
        colmax = jnp.max(G, axis=0, keepdims=True)
        E = jnp.exp(G - colmax)
        A = E / jnp.sum(E, axis=0, keepdims=True)
        upd = dot(A, V, (((0,), (0,)), ((), ())))
        vmean = jnp.mean(V, axis=0, keepdims=True)
        out_ref[:, sl] = jnp.broadcast_to(vmean, (L, D)) + dot(
            OH, upd - vmean, (((0,), (0,)), ((), ())))


def kernel(queries, keys, values, attn_mask, position_embedding_key, d_keys,
           u_W, u_b, v_W, v_b):
    del attn_mask, d_keys
    cnt_t = jnp.asarray(_CNT_T)
    rs = lambda x: x.reshape(B * L, H * D)
    big = pl.BlockSpec((L, HPS * D), lambda b, h2: (b, h2))
    const2 = lambda shape: pl.BlockSpec(shape, lambda b, h2: (0, 0))
    out = pl.pallas_call(
        _body,
        grid=(B, H // HPS),
        in_specs=[big, big, big, big,
                  const2((L, L)), const2((U, D)), const2((U, D)),
                  const2((1, U)), const2((1, U))],
        out_specs=big,
        out_shape=jax.ShapeDtypeStruct((B * L, H * D), jnp.float32),
        scratch_shapes=[pltpu.VMEM((HPS, U, L), jnp.float32)],
    )(rs(queries), rs(keys), rs(values), rs(position_embedding_key), cnt_t,
      u_W, v_W, u_b.reshape(1, U), v_b.reshape(1, U))
    return out.reshape(B, L, H, D)


# speedup vs baseline: 1.2324x; 1.2324x over previous
from functools import partial

import numpy as np
import jax
import jax.numpy as jnp
from jax import lax
from jax.experimental import pallas as pl
from jax.experimental.pallas import tpu as pltpu

B, L, H, D = 2, 2048, 12, 64
U = 40
SCALE = 0.125
NEG = -1e30
HPS = 2


def _threefry2x32(k1, k2, x0, x1):
    def rotl(x, d):
        return ((x << np.uint32(d)) | (x >> np.uint32(32 - d))).astype(np.uint32)
    x0 = x0.astype(np.uint32).copy()
    x1 = x1.astype(np.uint32).copy()
    ks = [np.uint32(k1), np.uint32(k2),
          np.uint32(np.uint32(k1) ^ np.uint32(k2) ^ np.uint32(0x1BD11BDA))]
    R = [(13, 15, 26, 6), (17, 29, 16, 24)]
    x0 = (x0 + ks[0]).astype(np.uint32)
    x1 = (x1 + ks[1]).astype(np.uint32)
    for i in range(5):
        for r in R[i % 2]:
            x0 = (x0 + x1).astype(np.uint32)
            x1 = rotl(x1, r) ^ x0
        x0 = (x0 + ks[(i + 1) % 3]).astype(np.uint32)
        x1 = (x1 + ks[(i + 2) % 3] + np.uint32(i + 1)).astype(np.uint32)
    return x0, x1


def _build_cnt_t() -> np.ndarray:
    b1, b2 = _threefry2x32(0, 42, np.zeros(2, np.uint32),
                           np.arange(2, dtype=np.uint32))
    lo1, lo2 = _threefry2x32(b1[1], b2[1], np.zeros(L * U, np.uint32),
                             np.arange(L * U, dtype=np.uint32))
    idx = ((lo1 ^ lo2) % np.uint32(L)).astype(np.int64).reshape(L, U)
    cnt_t = np.zeros((L, L), np.int8)
    np.add.at(cnt_t, (idx, np.broadcast_to(np.arange(L)[:, None], (L, U))), 1)
    return cnt_t


_CNT_T = _build_cnt_t()


def _body(q_ref, k_ref, v_ref, pek_ref, cnt_ref, uw_ref, vw_ref, ub_ref,
          vb_ref, out_ref, oh_ref):
    f32 = jnp.float32
    dot = partial(lax.dot_general, preferred_element_type=f32)
    cnt = cnt_ref[...].astype(f32)

    Ms = []
    for hh in range(HPS):
        sl = slice(hh * D, (hh + 1) * D)
        St = dot(k_ref[:, sl], q_ref[:, sl], (((1,), (1,)), ((), ())))
        smax = jnp.max(jnp.where(cnt > 0.0, St, NEG), axis=0, keepdims=True)
        ssum = jnp.sum(St * cnt, axis=0, keepdims=True)
        Ms.append(smax - ssum * (1.0 / L))
    M0 = jnp.concatenate(Ms, axis=0)

    iota = lax.broadcasted_iota(jnp.int32, (HPS, L), 1)
    Mv = M0
    for i in range(U):
        maxv = jnp.max(Mv, axis=1, keepdims=True)
        idx = jnp.min(jnp.where(Mv == maxv, iota, L), axis=1, keepdims=True)
        hit = iota == idx
        oh_ref[:, i, :] = hit.astype(f32)
        Mv = jnp.where(hit, NEG, Mv)

    bias = ub_ref[...] + vb_ref[...]
    for hh in range(HPS):
        sl = slice(hh * D, (hh + 1) * D)
        Q = q_ref[:, sl]
        K = k_ref[:, sl]
        V = v_ref[:, sl]
        PEK = pek_ref[:, sl]
        OH = oh_ref[hh]
        Qr = dot(OH, Q, (((1,), (0,)), ((), ())))
        G = dot(K + PEK, Qr, (((1,), (1,)), ((), ())))
        G = G + dot(K, uw_ref[...], (((1,), (1,)), ((), ())))
        G = G + dot(PEK, vw_ref[...], (((1,), (1,)), ((), ())))
        G = (G + bias) * SCALE
        colmax = jnp.max(G, axis=0, keepdims=True)
        E = jnp.exp(G - colmax)
        A = E / jnp.sum(E, axis=0, keepdims=True)
        upd = dot(A, V, (((0,), (0,)), ((), ())))
        vmean = jnp.mean(V, axis=0, keepdims=True)
        out_ref[:, sl] = jnp.broadcast_to(vmean, (L, D)) + dot(
            OH, upd - vmean, (((0,), (0,)), ((), ())))


def kernel(queries, keys, values, attn_mask, position_embedding_key, d_keys,
           u_W, u_b, v_W, v_b):
    del attn_mask, d_keys
    cnt_t = jnp.asarray(_CNT_T)
    rs = lambda x: x.reshape(B * L, H * D)
    big = pl.BlockSpec((L, HPS * D), lambda b, h2: (b, h2))
    const2 = lambda shape: pl.BlockSpec(shape, lambda b, h2: (0, 0))
    out = pl.pallas_call(
        _body,
        grid=(B, H // HPS),
        in_specs=[big, big, big, big,
                  const2((L, L)), const2((U, D)), const2((U, D)),
                  const2((1, U)), const2((1, U))],
        out_specs=big,
        out_shape=jax.ShapeDtypeStruct((B * L, H * D), jnp.float32),
        scratch_shapes=[pltpu.VMEM((HPS, U, L), jnp.float32)],
    )(rs(queries), rs(keys), rs(values), rs(position_embedding_key), cnt_t,
      u_W, v_W, u_b.reshape(1, U), v_b.reshape(1, U))
    return out.reshape(B, L, H, D)


# speedup vs baseline: 1.2991x; 1.0541x over previous
from functools import partial

import numpy as np
import jax
import jax.numpy as jnp
from jax import lax
from jax.experimental import pallas as pl
from jax.experimental.pallas import tpu as pltpu

B, L, H, D = 2, 2048, 12, 64
U = 40
SCALE = 0.125
NEG = -1e30
HPS = 2


def _threefry2x32(k1, k2, x0, x1):
    def rotl(x, d):
        return ((x << np.uint32(d)) | (x >> np.uint32(32 - d))).astype(np.uint32)
    x0 = x0.astype(np.uint32).copy()
    x1 = x1.astype(np.uint32).copy()
    ks = [np.uint32(k1), np.uint32(k2),
          np.uint32(np.uint32(k1) ^ np.uint32(k2) ^ np.uint32(0x1BD11BDA))]
    R = [(13, 15, 26, 6), (17, 29, 16, 24)]
    x0 = (x0 + ks[0]).astype(np.uint32)
    x1 = (x1 + ks[1]).astype(np.uint32)
    for i in range(5):
        for r in R[i % 2]:
            x0 = (x0 + x1).astype(np.uint32)
            x1 = rotl(x1, r) ^ x0
        x0 = (x0 + ks[(i + 1) % 3]).astype(np.uint32)
        x1 = (x1 + ks[(i + 2) % 3] + np.uint32(i + 1)).astype(np.uint32)
    return x0, x1


def _build_cnt_t() -> np.ndarray:
    b1, b2 = _threefry2x32(0, 42, np.zeros(2, np.uint32),
                           np.arange(2, dtype=np.uint32))
    lo1, lo2 = _threefry2x32(b1[1], b2[1], np.zeros(L * U, np.uint32),
                             np.arange(L * U, dtype=np.uint32))
    idx = ((lo1 ^ lo2) % np.uint32(L)).astype(np.int64).reshape(L, U)
    cnt_t = np.zeros((L, L), np.int8)
    np.add.at(cnt_t, (idx, np.broadcast_to(np.arange(L)[:, None], (L, U))), 1)
    return cnt_t


_CNT_T = _build_cnt_t()


def _body(q_hbm, k_hbm, v_hbm, pek_hbm, cnt_ref, uw_ref, vw_ref, ub_ref,
          vb_ref, out_ref, oh_ref, qv, kv, vv, pv, sems):
    f32 = jnp.float32
    dot = partial(lax.dot_general, preferred_element_type=f32)
    b = pl.program_id(0)
    h2 = pl.program_id(1)

    copies = []
    for hh in range(HPS):
        hg = h2 * HPS + hh
        for src, dst in ((q_hbm, qv), (k_hbm, kv), (v_hbm, vv),
                         (pek_hbm, pv)):
            cp = pltpu.make_async_copy(src.at[b, :, hg, :], dst.at[hh],
                                       sems.at[len(copies)])
            cp.start()
            copies.append(cp)

    cnt = cnt_ref[...].astype(f32)

    Ms = []
    for hh in range(HPS):
        copies[4 * hh + 0].wait()
        copies[4 * hh + 1].wait()
        St = dot(kv[hh], qv[hh], (((1,), (1,)), ((), ())))
        smax = jnp.max(jnp.where(cnt > 0.0, St, NEG), axis=0, keepdims=True)
        ssum = jnp.sum(St * cnt, axis=0, keepdims=True)
        Ms.append(smax - ssum * (1.0 / L))
    M0 = jnp.concatenate(Ms, axis=0)

    iota = lax.broadcasted_iota(jnp.int32, (HPS, L), 1)

    def topk_body(i, Mv):
        maxv = jnp.max(Mv, axis=1, keepdims=True)
        idx = jnp.min(jnp.where(Mv == maxv, iota, L), axis=1, keepdims=True)
        hit = iota == idx
        oh_ref[:, pl.ds(i, 1), :] = hit.astype(f32)[:, None, :]
        return jnp.where(hit, NEG, Mv)

    lax.fori_loop(0, U, topk_body, M0)

    bias = ub_ref[...] + vb_ref[...]
    for hh in range(HPS):
        copies[4 * hh + 2].wait()
        copies[4 * hh + 3].wait()
        sl = slice(hh * D, (hh + 1) * D)
        Q = qv[hh]
        K = kv[hh]
        V = vv[hh]
        PEK = pv[hh]
        OH = oh_ref[hh]
        Qr = dot(OH, Q, (((1,), (0,)), ((), ())))
        G = dot(K + PEK, Qr, (((1,), (1,)), ((), ())))
        G = G + dot(K, uw_ref[...], (((1,), (1,)), ((), ())))
        G = G + dot(PEK, vw_ref[...], (((1,), (1,)), ((), ())))
        G = (G + bias) * SCALE
        colmax = jnp.max(G, axis=0, keepdims=True)
        E = jnp.exp(G - colmax)
        A = E / jnp.sum(E, axis=0, keepdims=True)
        upd = dot(A, V, (((0,), (0,)), ((), ())))
        vmean = jnp.mean(V, axis=0, keepdims=True)
        out_ref[:, sl] = jnp.broadcast_to(vmean, (L, D)) + dot(
            OH, upd - vmean, (((0,), (0,)), ((), ())))


def kernel(queries, keys, values, attn_mask, position_embedding_key, d_keys,
           u_W, u_b, v_W, v_b):
    del attn_mask, d_keys
    cnt_t = jnp.asarray(_CNT_T)
    anyspec = pl.BlockSpec(memory_space=pl.ANY)
    const2 = lambda shape: pl.BlockSpec(shape, lambda b, h2: (0, 0))
    out = pl.pallas_call(
        _body,
        grid=(B, H // HPS),
        in_specs=[anyspec, anyspec, anyspec, anyspec,
                  const2((L, L)), const2((U, D)), const2((U, D)),
                  const2((1, U)), const2((1, U))],
        out_specs=pl.BlockSpec((L, HPS * D), lambda b, h2: (b, h2)),
        out_shape=jax.ShapeDtypeStruct((B * L, H * D), jnp.float32),
        scratch_shapes=[
            pltpu.VMEM((HPS, U, L), jnp.float32),
            pltpu.VMEM((HPS, L, D), jnp.float32),
            pltpu.VMEM((HPS, L, D), jnp.float32),
            pltpu.VMEM((HPS, L, D), jnp.float32),
            pltpu.VMEM((HPS, L, D), jnp.float32),
            pltpu.SemaphoreType.DMA((4 * HPS,)),
        ],
    )(queries, keys, values, position_embedding_key, cnt_t,
      u_W, v_W, u_b.reshape(1, U), v_b.reshape(1, U))
    return out.reshape(B, L, H, D)


# speedup vs baseline: 1.3096x; 1.0081x over previous
from functools import partial

import numpy as np
import jax
import jax.numpy as jnp
from jax import lax
from jax.experimental import pallas as pl
from jax.experimental.pallas import tpu as pltpu

B, L, H, D = 2, 2048, 12, 64
U = 40
SCALE = 0.125
NEG = -1e30
HPS = 2


def _threefry2x32(k1, k2, x0, x1):
    def rotl(x, d):
        return ((x << np.uint32(d)) | (x >> np.uint32(32 - d))).astype(np.uint32)
    x0 = x0.astype(np.uint32).copy()
    x1 = x1.astype(np.uint32).copy()
    ks = [np.uint32(k1), np.uint32(k2),
          np.uint32(np.uint32(k1) ^ np.uint32(k2) ^ np.uint32(0x1BD11BDA))]
    R = [(13, 15, 26, 6), (17, 29, 16, 24)]
    x0 = (x0 + ks[0]).astype(np.uint32)
    x1 = (x1 + ks[1]).astype(np.uint32)
    for i in range(5):
        for r in R[i % 2]:
            x0 = (x0 + x1).astype(np.uint32)
            x1 = rotl(x1, r) ^ x0
        x0 = (x0 + ks[(i + 1) % 3]).astype(np.uint32)
        x1 = (x1 + ks[(i + 2) % 3] + np.uint32(i + 1)).astype(np.uint32)
    return x0, x1


def _build_cnt_t() -> np.ndarray:
    b1, b2 = _threefry2x32(0, 42, np.zeros(2, np.uint32),
                           np.arange(2, dtype=np.uint32))
    lo1, lo2 = _threefry2x32(b1[1], b2[1], np.zeros(L * U, np.uint32),
                             np.arange(L * U, dtype=np.uint32))
    idx = ((lo1 ^ lo2) % np.uint32(L)).astype(np.int64).reshape(L, U)
    cnt_t = np.zeros((L, L), np.int8)
    np.add.at(cnt_t, (idx, np.broadcast_to(np.arange(L)[:, None], (L, U))), 1)
    return cnt_t


_CNT_T = _build_cnt_t()


NSTEP = B * (H // HPS)


def _copies(srcs, bufs, sems, slot, bn, h2n):
    cps = []
    for hh in range(HPS):
        hg = h2n * HPS + hh
        for j, (src, dst) in enumerate(zip(srcs, bufs)):
            cps.append(pltpu.make_async_copy(
                src.at[bn, :, hg, :], dst.at[slot, hh],
                sems.at[slot, 4 * hh + j]))
    return cps


def _body(q_hbm, k_hbm, v_hbm, pek_hbm, cnt_ref, uw_ref, vw_ref, ub_ref,
          vb_ref, out_ref, oh_ref, qv, kv, vv, pv, sems):
    f32 = jnp.float32
    dot = partial(lax.dot_general, preferred_element_type=f32)
    b = pl.program_id(0)
    h2 = pl.program_id(1)
    nh2 = H // HPS
    t = b * nh2 + h2
    slot = lax.rem(t, 2)
    srcs = (q_hbm, k_hbm, v_hbm, pek_hbm)
    bufs = (qv, kv, vv, pv)

    @pl.when(t == 0)
    def _():
        for cp in _copies(srcs, bufs, sems, 0, 0, 0):
            cp.start()

    @pl.when(t + 1 < NSTEP)
    def _():
        tn = t + 1
        for cp in _copies(srcs, bufs, sems, 1 - slot,
                          tn // nh2, lax.rem(tn, nh2)):
            cp.start()

    copies = _copies(srcs, bufs, sems, slot, b, h2)
    cnt = cnt_ref[...].astype(f32)

    Ms = []
    for hh in range(HPS):
        copies[4 * hh + 0].wait()
        copies[4 * hh + 1].wait()
        St = dot(kv[slot, hh], qv[slot, hh], (((1,), (1,)), ((), ())))
        smax = jnp.max(jnp.where(cnt > 0.0, St, NEG), axis=0, keepdims=True)
        ssum = jnp.sum(St * cnt, axis=0, keepdims=True)
        Ms.append(smax - ssum * (1.0 / L))
    M0 = jnp.concatenate(Ms, axis=0)

    iota = lax.broadcasted_iota(jnp.int32, (HPS, L), 1)

    def topk_body(i, Mv):
        maxv = jnp.max(Mv, axis=1, keepdims=True)
        idx = jnp.min(jnp.where(Mv == maxv, iota, L), axis=1, keepdims=True)
        hit = iota == idx
        oh_ref[:, pl.ds(i, 1), :] = hit.astype(f32)[:, None, :]
        return jnp.where(hit, NEG, Mv)

    lax.fori_loop(0, U, topk_body, M0)

    bias = ub_ref[...] + vb_ref[...]
    for hh in range(HPS):
        copies[4 * hh + 2].wait()
        copies[4 * hh + 3].wait()
        sl = slice(hh * D, (hh + 1) * D)
        Q = qv[slot, hh]
        K = kv[slot, hh]
        V = vv[slot, hh]
        PEK = pv[slot, hh]
        OH = oh_ref[hh]
        Qr = dot(OH, Q, (((1,), (0,)), ((), ())))
        G = dot(K + PEK, Qr, (((1,), (1,)), ((), ())))
        G = G + dot(K, uw_ref[...], (((1,), (1,)), ((), ())))
        G = G + dot(PEK, vw_ref[...], (((1,), (1,)), ((), ())))
        G = (G + bias) * SCALE
        colmax = jnp.max(G, axis=0, keepdims=True)
        E = jnp.exp(G - colmax)
        A = E / jnp.sum(E, axis=0, keepdims=True)
        upd = dot(A, V, (((0,), (0,)), ((), ())))
        vmean = jnp.mean(V, axis=0, keepdims=True)
        out_ref[:, sl] = jnp.broadcast_to(vmean, (L, D)) + dot(
            OH, upd - vmean, (((0,), (0,)), ((), ())))


def kernel(queries, keys, values, attn_mask, position_embedding_key, d_keys,
           u_W, u_b, v_W, v_b):
    del attn_mask, d_keys
    cnt_t = jnp.asarray(_CNT_T)
    anyspec = pl.BlockSpec(memory_space=pl.ANY)
    const2 = lambda shape: pl.BlockSpec(shape, lambda b, h2: (0, 0))
    out = pl.pallas_call(
        _body,
        grid=(B, H // HPS),
        in_specs=[anyspec, anyspec, anyspec, anyspec,
                  const2((L, L)), const2((U, D)), const2((U, D)),
                  const2((1, U)), const2((1, U))],
        out_specs=pl.BlockSpec((L, HPS * D), lambda b, h2: (b, h2)),
        out_shape=jax.ShapeDtypeStruct((B * L, H * D), jnp.float32),
        scratch_shapes=[
            pltpu.VMEM((HPS, U, L), jnp.float32),
            pltpu.VMEM((2, HPS, L, D), jnp.float32),
            pltpu.VMEM((2, HPS, L, D), jnp.float32),
            pltpu.VMEM((2, HPS, L, D), jnp.float32),
            pltpu.VMEM((2, HPS, L, D), jnp.float32),
            pltpu.SemaphoreType.DMA((2, 4 * HPS)),
        ],
    )(queries, keys, values, position_embedding_key, cnt_t,
      u_W, v_W, u_b.reshape(1, U), v_b.reshape(1, U))
    return out.reshape(B, L, H, D)
